# baseline (speedup 1.0000x reference)
import functools

import jax
import jax.numpy as jnp
from jax import lax
from jax.experimental import pallas as pl
from jax.experimental.pallas import tpu as pltpu
from jax.experimental.pallas import tpu_sc as plsc

_HEIGHT = 8
_TOPICS = 64
_NTIME = 24
_BATCH = 16384
_NC = 2
_NS = 16
_NW = _NC * _NS
_BPW = _BATCH // _NW
_CHUNK = 128
_NCHUNK = _BPW // _CHUNK
_LANES = 16


def _stage_indices(loc_hbm, t_hbm, g7_hbm, loc_v, t_v, c7_v, sem, rb):
    pltpu.sync_copy(loc_hbm.at[pl.ds(rb, _NCHUNK)], loc_v)
    pltpu.sync_copy(t_hbm.at[pl.ds(rb, _NCHUNK)], t_v)
    cps = [
        pltpu.async_copy(g7_hbm.at[loc_v.at[j]], c7_v.at[j], sem)
        for j in range(_NCHUNK)
    ]
    for cp in cps:
        cp.wait()


def _fill_ridx(levels, t_v, c7_v, ridx_v):
    for j in range(_NCHUNK):
        def ridx_body(v, _, j=j):
            s = pl.ds(v * _LANES, _LANES)
            c7 = c7_v[j, s]
            t = t_v[j, s]
            li = lax.shift_right_logical(c7, 7)
            lo = lax.bitwise_and(c7, 127)
            for i, h in enumerate(levels):
                if h == 0:
                    ridx_v[i, j, s] = t
                else:
                    sh = 7 - h
                    cell = lax.bitwise_or(
                        lax.shift_left(lax.shift_right_logical(li, sh), h),
                        lax.shift_right_logical(lo, sh))
                    ridx_v[i, j, s] = cell * _NTIME + t
            return 0
        lax.fori_loop(0, _CHUNK // _LANES, ridx_body, 0)


def _body_low(loc_hbm, t_hbm, g7_hbm, p0, p1, p2, p3, p4, p5,
              out_hbm, loc_v, t_v, c7_v, ridx_v, bufs_v, sem):
    params = (p0, p1, p2, p3, p4, p5)
    nlev = len(params)
    wid = lax.axis_index("s") * _NC + lax.axis_index("c")
    rb = wid * _NCHUNK

    _stage_indices(loc_hbm, t_hbm, g7_hbm, loc_v, t_v, c7_v, sem, rb)
    _fill_ridx(tuple(range(nlev)), t_v, c7_v, ridx_v)

    for j in range(_NCHUNK):
        cps = [
            pltpu.async_copy(params[h].at[ridx_v.at[h, j]], bufs_v.at[h], sem)
            for h in range(nlev)
        ]
        for cp in cps:
            cp.wait()

        def acc_body(r, _):
            for c in range(_TOPICS // _LANES):
                s = pl.ds(c * _LANES, _LANES)
                x = bufs_v[0, r, s]
                for h in range(1, nlev):
                    x = x + bufs_v[h, r, s]
                bufs_v[0, r, s] = x
            return 0
        lax.fori_loop(0, _CHUNK, acc_body, 0)

        pltpu.sync_copy(bufs_v.at[0],
                        out_hbm.at[pl.ds(wid * _BPW + j * _CHUNK, _CHUNK)])


def _body_high(loc_hbm, t_hbm, g7_hbm, part_hbm, p6, p7,
               out_hbm, loc_v, t_v, c7_v, ridx_v, bufs_v, part_v, sem):
    params = (p6, p7)
    wid = lax.axis_index("s") * _NC + lax.axis_index("c")
    rb = wid * _NCHUNK

    _stage_indices(loc_hbm, t_hbm, g7_hbm, loc_v, t_v, c7_v, sem, rb)
    _fill_ridx((6, 7), t_v, c7_v, ridx_v)

    for j in range(_NCHUNK):
        base = wid * _BPW + j * _CHUNK
        cps = [
            pltpu.async_copy(params[h].at[ridx_v.at[h, j]], bufs_v.at[h], sem)
            for h in range(2)
        ]
        cps.append(pltpu.async_copy(part_hbm.at[pl.ds(base, _CHUNK)],
                                    part_v, sem))
        for cp in cps:
            cp.wait()

        iot = lax.iota(jnp.int32, _LANES)

        def acc_body(r, _):
            rows = iot * 0 + r
            for g in range(_TOPICS // (2 * _LANES)):
                s = pl.ds(g * 2 * _LANES, 2 * _LANES)
                e6, o6 = plsc.unpack(bufs_v[0, r, s],
                                     format=plsc.PackFormat.INTERLEAVED,
                                     preferred_element_type=jnp.float32)
                e7, o7 = plsc.unpack(bufs_v[1, r, s],
                                     format=plsc.PackFormat.INTERLEAVED,
                                     preferred_element_type=jnp.float32)
                cols = g * 2 * _LANES + 2 * iot
                plsc.addupdate_scatter(part_v, [rows, cols], e6 + e7)
                plsc.addupdate_scatter(part_v, [rows, cols + 1], o6 + o7)
            return 0
        lax.fori_loop(0, _CHUNK, acc_body, 0)

        pltpu.sync_copy(part_v, out_hbm.at[pl.ds(base, _CHUNK)])


def kernel(location_indices, time_slices, grid_assign,
           param_0, param_1, param_2, param_3,
           param_4, param_5, param_6, param_7):
    loc2 = location_indices.astype(jnp.int32).reshape(_BATCH // _CHUNK, _CHUNK)
    t2 = time_slices.astype(jnp.int32).reshape(_BATCH // _CHUNK, _CHUNK)
    g7 = grid_assign[_HEIGHT - 1].astype(jnp.int32)
    low = [p.reshape(-1, _TOPICS) for p in
           (param_0, param_1, param_2, param_3, param_4, param_5)]
    high = [p.astype(jnp.bfloat16).reshape(-1, _TOPICS)
            for p in (param_6, param_7)]

    mesh = plsc.VectorSubcoreMesh(core_axis_name="c", subcore_axis_name="s")
    cparams = pltpu.CompilerParams(use_tc_tiling_on_sc=False,
                                   needs_layout_passes=False)
    out_ty = jax.ShapeDtypeStruct((_BATCH, _TOPICS), jnp.float32)
    idx_scr = [
        pltpu.VMEM((_NCHUNK, _CHUNK), jnp.int32),
        pltpu.VMEM((_NCHUNK, _CHUNK), jnp.int32),
        pltpu.VMEM((_NCHUNK, _CHUNK), jnp.int32),
    ]

    run_low = functools.partial(
        pl.kernel, mesh=mesh, compiler_params=cparams, out_type=out_ty,
        scratch_types=idx_scr + [
            pltpu.VMEM((6, _NCHUNK, _CHUNK), jnp.int32),
            pltpu.VMEM((6, _CHUNK, _TOPICS), jnp.float32),
            pltpu.SemaphoreType.DMA,
        ],
    )(_body_low)
    part = run_low(loc2, t2, g7, *low)

    run_high = functools.partial(
        pl.kernel, mesh=mesh, compiler_params=cparams, out_type=out_ty,
        scratch_types=idx_scr + [
            pltpu.VMEM((2, _NCHUNK, _CHUNK), jnp.int32),
            pltpu.VMEM((2, _CHUNK, _TOPICS), jnp.bfloat16),
            pltpu.VMEM((_CHUNK, _TOPICS), jnp.float32),
            pltpu.SemaphoreType.DMA,
        ],
    )(_body_high)
    return run_high(loc2, t2, g7, part, *high)


# speedup vs baseline: 1.2508x; 1.2508x over previous
import functools

import jax
import jax.numpy as jnp
from jax import lax
from jax.experimental import pallas as pl
from jax.experimental.pallas import tpu as pltpu
from jax.experimental.pallas import tpu_sc as plsc

_HEIGHT = 8
_TOPICS = 64
_NTIME = 24
_BATCH = 16384
_NC = 2
_NS = 16
_NW = _NC * _NS
_BPW = _BATCH // _NW
_CHUNK = 128
_NCHUNK = _BPW // _CHUNK
_LANES = 16


def _stage_indices(loc_hbm, t_hbm, g7_hbm, loc_v, t_v, c7_v, sem, rb):
    pltpu.sync_copy(loc_hbm.at[pl.ds(rb, _NCHUNK)], loc_v)
    pltpu.sync_copy(t_hbm.at[pl.ds(rb, _NCHUNK)], t_v)
    cps = [
        pltpu.async_copy(g7_hbm.at[loc_v.at[j]], c7_v.at[j], sem)
        for j in range(_NCHUNK)
    ]
    for cp in cps:
        cp.wait()


def _fill_ridx(levels, t_v, c7_v, ridx_v):
    for j in range(_NCHUNK):
        def ridx_body(v, _, j=j):
            s = pl.ds(v * _LANES, _LANES)
            c7 = c7_v[j, s]
            t = t_v[j, s]
            li = lax.shift_right_logical(c7, 7)
            lo = lax.bitwise_and(c7, 127)
            for i, h in enumerate(levels):
                if h == 0:
                    ridx_v[i, j, s] = t
                else:
                    sh = 7 - h
                    cell = lax.bitwise_or(
                        lax.shift_left(lax.shift_right_logical(li, sh), h),
                        lax.shift_right_logical(lo, sh))
                    ridx_v[i, j, s] = cell * _NTIME + t
            return 0
        lax.fori_loop(0, _CHUNK // _LANES, ridx_body, 0)


def _body_low(loc_hbm, t_hbm, g7_hbm, p0, p1, p2, p3, p4, p5,
              out_hbm, loc_v, t_v, c7_v, ridx_v, bufs_v, sem):
    params = (p0, p1, p2, p3, p4, p5)
    nlev = len(params)
    wid = lax.axis_index("s") * _NC + lax.axis_index("c")
    rb = wid * _NCHUNK

    _stage_indices(loc_hbm, t_hbm, g7_hbm, loc_v, t_v, c7_v, sem, rb)
    _fill_ridx(tuple(range(nlev)), t_v, c7_v, ridx_v)

    for j in range(_NCHUNK):
        cps = [
            pltpu.async_copy(params[h].at[ridx_v.at[h, j]], bufs_v.at[h], sem)
            for h in range(nlev)
        ]
        for cp in cps:
            cp.wait()

        def acc_body(r, _):
            for c in range(_TOPICS // _LANES):
                s = pl.ds(c * _LANES, _LANES)
                x = bufs_v[0, r, s]
                for h in range(1, nlev):
                    x = x + bufs_v[h, r, s]
                bufs_v[0, r, s] = x
            return 0
        lax.fori_loop(0, _CHUNK, acc_body, 0)

        pltpu.sync_copy(bufs_v.at[0],
                        out_hbm.at[pl.ds(wid * _BPW + j * _CHUNK, _CHUNK)])


def _make_body_add(level):
    def _body_add(loc_hbm, t_hbm, g7_hbm, part_hbm, ph,
                  out_hbm, loc_v, t_v, c7_v, ridx_v, bufs_v, part_v, sem):
        wid = lax.axis_index("s") * _NC + lax.axis_index("c")
        rb = wid * _NCHUNK

        _stage_indices(loc_hbm, t_hbm, g7_hbm, loc_v, t_v, c7_v, sem, rb)
        _fill_ridx((level,), t_v, c7_v, ridx_v)

        for j in range(_NCHUNK):
            base = wid * _BPW + j * _CHUNK
            cps = [
                pltpu.async_copy(ph.at[ridx_v.at[0, j]], bufs_v, sem),
                pltpu.async_copy(part_hbm.at[pl.ds(base, _CHUNK)],
                                 part_v, sem),
            ]
            for cp in cps:
                cp.wait()

            def acc_body(r, _):
                for c in range(_TOPICS // _LANES):
                    s = pl.ds(c * _LANES, _LANES)
                    part_v[r, s] = part_v[r, s] + bufs_v[r, s]
                return 0
            lax.fori_loop(0, _CHUNK, acc_body, 0)

            pltpu.sync_copy(part_v, out_hbm.at[pl.ds(base, _CHUNK)])
    return _body_add


def kernel(location_indices, time_slices, grid_assign,
           param_0, param_1, param_2, param_3,
           param_4, param_5, param_6, param_7):
    loc2 = location_indices.astype(jnp.int32).reshape(_BATCH // _CHUNK, _CHUNK)
    t2 = time_slices.astype(jnp.int32).reshape(_BATCH // _CHUNK, _CHUNK)
    g7 = grid_assign[_HEIGHT - 1].astype(jnp.int32)
    low = [p.reshape(-1, _TOPICS) for p in
           (param_0, param_1, param_2, param_3, param_4, param_5)]
    high = [p.reshape(-1, _TOPICS) for p in (param_6, param_7)]

    mesh = plsc.VectorSubcoreMesh(core_axis_name="c", subcore_axis_name="s")
    cparams = pltpu.CompilerParams(use_tc_tiling_on_sc=False)
    out_ty = jax.ShapeDtypeStruct((_BATCH, _TOPICS), jnp.float32)
    idx_scr = [
        pltpu.VMEM((_NCHUNK, _CHUNK), jnp.int32),
        pltpu.VMEM((_NCHUNK, _CHUNK), jnp.int32),
        pltpu.VMEM((_NCHUNK, _CHUNK), jnp.int32),
    ]

    run_low = functools.partial(
        pl.kernel, mesh=mesh, compiler_params=cparams, out_type=out_ty,
        scratch_types=idx_scr + [
            pltpu.VMEM((6, _NCHUNK, _CHUNK), jnp.int32),
            pltpu.VMEM((6, _CHUNK, _TOPICS), jnp.float32),
            pltpu.SemaphoreType.DMA,
        ],
    )(_body_low)
    part = run_low(loc2, t2, g7, *low)

    add_scr = idx_scr + [
        pltpu.VMEM((1, _NCHUNK, _CHUNK), jnp.int32),
        pltpu.VMEM((_CHUNK, _TOPICS), jnp.float32),
        pltpu.VMEM((_CHUNK, _TOPICS), jnp.float32),
        pltpu.SemaphoreType.DMA,
    ]
    for lev, ph in zip((6, 7), high):
        run_add = functools.partial(
            pl.kernel, mesh=mesh, compiler_params=cparams, out_type=out_ty,
            scratch_types=add_scr,
        )(_make_body_add(lev))
        part = run_add(loc2, t2, g7, part, ph)
    return part


# speedup vs baseline: 1.4241x; 1.1385x over previous
import functools

import jax
import jax.numpy as jnp
from jax import lax
from jax.experimental import pallas as pl
from jax.experimental.pallas import tpu as pltpu
from jax.experimental.pallas import tpu_sc as plsc

_HEIGHT = 8
_TOPICS = 64
_NTIME = 24
_BATCH = 16384
_NC = 2
_NS = 16
_NW = _NC * _NS
_BPW = _BATCH // _NW
_CHUNK = 128
_NCHUNK = _BPW // _CHUNK
_LANES = 16


def _stage_indices(loc_hbm, t_hbm, g7_hbm, loc_v, t_v, c7_v, sem, rb):
    pltpu.sync_copy(loc_hbm.at[pl.ds(rb, _NCHUNK)], loc_v)
    pltpu.sync_copy(t_hbm.at[pl.ds(rb, _NCHUNK)], t_v)
    cps = [
        pltpu.async_copy(g7_hbm.at[loc_v.at[j]], c7_v.at[j], sem)
        for j in range(_NCHUNK)
    ]
    for cp in cps:
        cp.wait()


_REPS = (32, 16, 4, 1, 1, 1, 1, 1)


def _fill_ridx(levels, t_v, c7_v, ridx_v):
    for j in range(_NCHUNK):
        def ridx_body(v, _, j=j):
            s = pl.ds(v * _LANES, _LANES)
            c7 = c7_v[j, s]
            t = t_v[j, s]
            li = lax.shift_right_logical(c7, 7)
            lo = lax.bitwise_and(c7, 127)
            for i, h in enumerate(levels):
                if h == 0:
                    row = t
                else:
                    sh = 7 - h
                    cell = lax.bitwise_or(
                        lax.shift_left(lax.shift_right_logical(li, sh), h),
                        lax.shift_right_logical(lo, sh))
                    row = cell * _NTIME + t
                reps = _REPS[h]
                if reps > 1:
                    rep = lax.bitwise_and(lo, reps - 1)
                    row = row + rep * ((4 ** h) * _NTIME)
                ridx_v[i, j, s] = row
            return 0
        lax.fori_loop(0, _CHUNK // _LANES, ridx_body, 0)


def _body_low(loc_hbm, t_hbm, g7_hbm, p0, p1, p2, p3, p4, p5,
              out_hbm, loc_v, t_v, c7_v, ridx_v, bufs_v, sem):
    params = (p0, p1, p2, p3, p4, p5)
    nlev = len(params)
    wid = lax.axis_index("s") * _NC + lax.axis_index("c")
    rb = wid * _NCHUNK

    _stage_indices(loc_hbm, t_hbm, g7_hbm, loc_v, t_v, c7_v, sem, rb)
    _fill_ridx(tuple(range(nlev)), t_v, c7_v, ridx_v)

    for j in range(_NCHUNK):
        cps = [
            pltpu.async_copy(params[h].at[ridx_v.at[h, j]], bufs_v.at[h], sem)
            for h in range(nlev)
        ]
        for cp in cps:
            cp.wait()

        def acc_body(r, _):
            for c in range(_TOPICS // _LANES):
                s = pl.ds(c * _LANES, _LANES)
                x = bufs_v[0, r, s]
                for h in range(1, nlev):
                    x = x + bufs_v[h, r, s]
                bufs_v[0, r, s] = x
            return 0
        lax.fori_loop(0, _CHUNK, acc_body, 0)

        pltpu.sync_copy(bufs_v.at[0],
                        out_hbm.at[pl.ds(wid * _BPW + j * _CHUNK, _CHUNK)])


def _body_high(loc_hbm, t_hbm, g7_hbm, part_hbm, p6, p7,
               out_hbm, loc_v, t_v, c7_v, ridx_v, bufs_v, part_v, sem):
    params = (p6, p7)
    wid = lax.axis_index("s") * _NC + lax.axis_index("c")
    rb = wid * _NCHUNK

    _stage_indices(loc_hbm, t_hbm, g7_hbm, loc_v, t_v, c7_v, sem, rb)
    _fill_ridx((6, 7), t_v, c7_v, ridx_v)

    for j in range(_NCHUNK):
        base = wid * _BPW + j * _CHUNK
        cps = [
            pltpu.async_copy(params[h].at[ridx_v.at[h, j]], bufs_v.at[h], sem)
            for h in range(2)
        ]
        cps.append(pltpu.async_copy(part_hbm.at[pl.ds(base, _CHUNK)],
                                    part_v, sem))
        for cp in cps:
            cp.wait()

        def acc_body(r, _):
            for c in range(_TOPICS // _LANES):
                s = pl.ds(c * _LANES, _LANES)
                part_v[r, s] = part_v[r, s] + bufs_v[0, r, s] + bufs_v[1, r, s]
            return 0
        lax.fori_loop(0, _CHUNK, acc_body, 0)

        pltpu.sync_copy(part_v, out_hbm.at[pl.ds(base, _CHUNK)])


def kernel(location_indices, time_slices, grid_assign,
           param_0, param_1, param_2, param_3,
           param_4, param_5, param_6, param_7):
    loc2 = location_indices.astype(jnp.int32).reshape(_BATCH // _CHUNK, _CHUNK)
    t2 = time_slices.astype(jnp.int32).reshape(_BATCH // _CHUNK, _CHUNK)
    g7 = grid_assign[_HEIGHT - 1].astype(jnp.int32)
    low = [jnp.tile(p.reshape(-1, _TOPICS), (_REPS[h], 1))
           if _REPS[h] > 1 else p.reshape(-1, _TOPICS)
           for h, p in enumerate(
               (param_0, param_1, param_2, param_3, param_4, param_5))]
    high = [p.reshape(-1, _TOPICS) for p in (param_6, param_7)]

    mesh = plsc.VectorSubcoreMesh(core_axis_name="c", subcore_axis_name="s")
    cparams = pltpu.CompilerParams(use_tc_tiling_on_sc=False)
    out_ty = jax.ShapeDtypeStruct((_BATCH, _TOPICS), jnp.float32)
    idx_scr = [
        pltpu.VMEM((_NCHUNK, _CHUNK), jnp.int32),
        pltpu.VMEM((_NCHUNK, _CHUNK), jnp.int32),
        pltpu.VMEM((_NCHUNK, _CHUNK), jnp.int32),
    ]

    run_low = functools.partial(
        pl.kernel, mesh=mesh, compiler_params=cparams, out_type=out_ty,
        scratch_types=idx_scr + [
            pltpu.VMEM((6, _NCHUNK, _CHUNK), jnp.int32),
            pltpu.VMEM((6, _CHUNK, _TOPICS), jnp.float32),
            pltpu.SemaphoreType.DMA,
        ],
    )(_body_low)
    part = run_low(loc2, t2, g7, *low)

    run_high = functools.partial(
        pl.kernel, mesh=mesh, compiler_params=cparams, out_type=out_ty,
        scratch_types=idx_scr + [
            pltpu.VMEM((2, _NCHUNK, _CHUNK), jnp.int32),
            pltpu.VMEM((2, _CHUNK, _TOPICS), jnp.float32),
            pltpu.VMEM((_CHUNK, _TOPICS), jnp.float32),
            pltpu.SemaphoreType.DMA,
        ],
    )(_body_high)
    return run_high(loc2, t2, g7, part, *high)
